# baseline (speedup 1.0000x reference)
import functools

import jax
import jax.numpy as jnp
from jax import lax
from jax.experimental import pallas as pl
from jax.experimental.pallas import tpu as pltpu
from jax.experimental.pallas import tpu_sc as plsc

N = 10000
E = 320000
D = 128
NC = 2
NS = 16
NW = NC * NS
NPAD = 10240
RPT = NPAD // NS
NBUF = 3
CHUNK = 96
K0 = 105
K1 = 105
EPAD = NS * (K0 + K1) * CHUNK
RB = 2000
GRID = N // RB

_mesh = plsc.VectorSubcoreMesh(core_axis_name="c", subcore_axis_name="s")


@functools.partial(
    pl.kernel,
    out_type=jax.ShapeDtypeStruct((NC, NPAD), jnp.float32),
    mesh=_mesh,
    scratch_types=[
        pltpu.VMEM((K0 * CHUNK,), jnp.int32),
        pltpu.VMEM((K0, CHUNK), jnp.int32),
        pltpu.VMEM((CHUNK,), jnp.float32),
        pltpu.VMEM_SHARED((NPAD,), jnp.float32),
        pltpu.SemaphoreType.DMA,
        pltpu.SemaphoreType.DMA,
    ],
)
def _deg(packed_hbm, pads_hbm, zeros_hbm, out_hbm,
         packed_v, dst2d, ones_v, acc, zsem, ssem):
    c = lax.axis_index("c")
    s = lax.axis_index("s")
    pltpu.async_copy(zeros_hbm, acc.at[pl.ds(s * RPT, RPT)], zsem)
    _stage_packed(packed_hbm, pads_hbm, packed_v, c, s, ssem)
    for i in range(CHUNK // 16):
        ones_v[pl.ds(i * 16, 16)] = jnp.ones((16,), jnp.float32)
    for j in range(K0):
        for i in range(CHUNK // 16):
            v = packed_v[pl.ds(j * CHUNK + i * 16, 16)]
            dst2d[j, pl.ds(i * 16, 16)] = lax.shift_right_logical(v, 16)
    pltpu.make_async_copy(zeros_hbm, acc.at[pl.ds(s * RPT, RPT)], zsem).wait()
    plsc.subcore_barrier()

    def fire(j, _):
        pltpu.async_copy(ones_v, acc.at[dst2d.at[j]], ssem, add=True)
        return ()

    lax.fori_loop(0, K0, fire, ())

    def drain(j, _):
        pltpu.make_async_copy(ones_v, acc.at[dst2d.at[j]], ssem).wait()
        return ()

    lax.fori_loop(0, K0, drain, ())
    plsc.subcore_barrier()
    pltpu.sync_copy(acc.at[pl.ds(s * RPT, RPT)],
                    out_hbm.at[c, pl.ds(s * RPT, RPT)])


def _stage_packed(packed_hbm, pads_hbm, packed_v, c, s, sem):
    n = K0 * CHUNK

    @pl.when(jnp.logical_or(c == 0, s < NS - 1))
    def _():
        base = jnp.where(c == 0, 0, NS * K0 * CHUNK)
        pltpu.async_copy(packed_hbm.at[pl.ds(base + s * n, n)],
                         packed_v.at[pl.ds(0, n)], sem)
        pltpu.make_async_copy(packed_hbm.at[pl.ds(base + s * n, n)],
                              packed_v.at[pl.ds(0, n)], sem).wait()

    @pl.when(jnp.logical_and(c == 1, s == NS - 1))
    def _():
        nreal = E - (2 * NS - 1) * n
        pltpu.async_copy(packed_hbm.at[pl.ds(E - nreal, nreal)],
                         packed_v.at[pl.ds(0, nreal)], sem)
        pltpu.make_async_copy(packed_hbm.at[pl.ds(E - nreal, nreal)],
                              packed_v.at[pl.ds(0, nreal)], sem).wait()
        pltpu.async_copy(pads_hbm, packed_v.at[pl.ds(nreal, EPAD - E)], sem)
        pltpu.make_async_copy(pads_hbm,
                              packed_v.at[pl.ds(nreal, EPAD - E)], sem).wait()


@functools.partial(
    pl.kernel,
    out_type=jax.ShapeDtypeStruct((NC, NPAD, D), jnp.float32),
    mesh=_mesh,
    scratch_types=(
        [pltpu.VMEM((K0 * CHUNK,), jnp.int32)]
        + [pltpu.VMEM((CHUNK,), jnp.int32) for _ in range(2 * NBUF)]
        + [pltpu.VMEM((CHUNK, D), jnp.float32) for _ in range(NBUF)]
        + [pltpu.VMEM_SHARED((NPAD, D), jnp.float32)]
        + [pltpu.SemaphoreType.DMA for _ in range(2 * NBUF)]
    ),
)
def _agg(packed_hbm, pads_hbm, h_hbm, zeros_hbm, out_hbm, packed_v, *rest):
    sidx = rest[0:NBUF]
    didx = rest[NBUF:2 * NBUF]
    bufs = rest[2 * NBUF:3 * NBUF]
    acc = rest[3 * NBUF]
    gsems = rest[3 * NBUF + 1:4 * NBUF + 1]
    ssems = rest[4 * NBUF + 1:5 * NBUF + 1]
    c = lax.axis_index("c")
    s = lax.axis_index("s")
    pltpu.async_copy(zeros_hbm, acc.at[pl.ds(s * RPT, RPT)], gsems[0])
    _stage_packed(packed_hbm, pads_hbm, packed_v, c, s, gsems[1])
    pltpu.make_async_copy(zeros_hbm, acc.at[pl.ds(s * RPT, RPT)],
                          gsems[0]).wait()
    plsc.subcore_barrier()

    def unpack(j, sb, db):
        for i in range(CHUNK // 16):
            v = packed_v[pl.ds(j * CHUNK + i * 16, 16)]
            sb[pl.ds(i * 16, 16)] = lax.bitwise_and(v, 0xFFFF)
            db[pl.ds(i * 16, 16)] = lax.shift_right_logical(v, 16)

    def run(nch):
        for b in range(NBUF):
            unpack(b, sidx[b], didx[b])
            pltpu.async_copy(h_hbm.at[sidx[b]], bufs[b], gsems[b])

        def outer(it, _):
            jj = it * NBUF
            for b in range(NBUF):
                pltpu.make_async_copy(h_hbm.at[sidx[b]], bufs[b],
                                      gsems[b]).wait()
                pltpu.async_copy(bufs[b], acc.at[didx[b]], ssems[b], add=True)
            for b in range(NBUF):
                nxt = jj + b + NBUF
                pltpu.make_async_copy(bufs[b], acc.at[didx[b]],
                                      ssems[b]).wait()

                @pl.when(nxt < nch)
                def _():
                    unpack(nxt, sidx[b], didx[b])
                    pltpu.async_copy(h_hbm.at[sidx[b]], bufs[b], gsems[b])

            return ()

        lax.fori_loop(0, nch // NBUF, outer, ())

    @pl.when(c == 0)
    def _():
        run(K0)

    @pl.when(c == 1)
    def _():
        run(K1)

    plsc.subcore_barrier()
    pltpu.sync_copy(acc.at[pl.ds(s * RPT, RPT)],
                    out_hbm.at[c, pl.ds(s * RPT, RPT)])


def _dinv_body(degp_ref, dinv_ref):
    dp = degp_ref[...]
    d = dp[:NPAD] + dp[NPAD:] + 1.0
    dinv_ref[...] = lax.rsqrt(d)


def _mm_scale_body(x_ref, w_ref, dinv_ref, o_ref):
    h = jnp.dot(x_ref[...], w_ref[...], preferred_element_type=jnp.float32)
    o_ref[...] = h * dinv_ref[...]


def _combine_stats_body(ap_ref, h_ref, dinv_ref, b_ref, o_ref, s1_ref, s2_ref):
    i = pl.program_id(0)
    ap = ap_ref[...]
    o = (ap[0] + ap[1] + h_ref[...]) * dinv_ref[...] + b_ref[...]
    o_ref[...] = o

    @pl.when(i == 0)
    def _():
        s1_ref[...] = jnp.zeros_like(s1_ref)
        s2_ref[...] = jnp.zeros_like(s2_ref)

    s1_ref[...] += jnp.sum(o, axis=0, keepdims=True)
    s2_ref[...] += jnp.sum(o * o, axis=0, keepdims=True)


def _bn_mm_body(o_ref, s1_ref, s2_ref, g_ref, be_ref, w_ref, dinv_ref, h_ref):
    mean = s1_ref[...] / N
    var = s2_ref[...] / N - mean * mean
    rstd = lax.rsqrt(var + 1e-5)
    y = (o_ref[...] - mean) * (rstd * g_ref[...]) + be_ref[...]
    y = jnp.maximum(y, 0.0)
    h = jnp.dot(y, w_ref[...], preferred_element_type=jnp.float32)
    h_ref[...] = h * dinv_ref[...]


def _final_body(ap_ref, h_ref, dinv_ref, b_ref, o_ref):
    ap = ap_ref[...]
    o_ref[...] = (ap[0] + ap[1] + h_ref[...]) * dinv_ref[...] + b_ref[...]


def kernel(x, edge_index, W1, b1, W2, b2, gamma, beta):
    f32 = jnp.float32
    src = edge_index[0].astype(jnp.int32)
    dst = edge_index[1].astype(jnp.int32)
    npad_e = EPAD - E
    pad_src = jnp.arange(npad_e, dtype=jnp.int32) % N
    pad_dst = N + jnp.arange(npad_e, dtype=jnp.int32) % (NPAD - N)
    pads = jnp.bitwise_or(pad_src, pad_dst << 16)
    packed_a = jnp.bitwise_or(src, dst << 16)
    z1 = jnp.zeros((RPT,), f32)
    z2 = jnp.zeros((RPT, D), f32)
    b1r = b1.reshape(1, D)
    b2r = b2.reshape(1, D)
    gr = gamma.reshape(1, D)
    ber = beta.reshape(1, D)

    deg_p = _deg(packed_a, pads, z1)

    dinv = pl.pallas_call(
        _dinv_body,
        out_shape=jax.ShapeDtypeStruct((NPAD, 1), f32),
    )(deg_p.reshape(NC * NPAD, 1))

    row_spec = pl.BlockSpec((RB, D), lambda i: (i, 0))
    vec_spec = pl.BlockSpec((RB, 1), lambda i: (i, 0))
    full_spec = pl.BlockSpec((1, D), lambda i: (0, 0))
    w_spec = pl.BlockSpec((D, D), lambda i: (0, 0))
    part_spec = pl.BlockSpec((NC, RB, D), lambda i: (0, i, 0))

    h1 = pl.pallas_call(
        _mm_scale_body,
        grid=(GRID,),
        in_specs=[row_spec, w_spec, vec_spec],
        out_specs=row_spec,
        out_shape=jax.ShapeDtypeStruct((N, D), f32),
    )(x, W1, dinv)

    agg1 = _agg(packed_a, pads, h1, z2)

    out1, s1, s2 = pl.pallas_call(
        _combine_stats_body,
        grid=(GRID,),
        in_specs=[part_spec, row_spec, vec_spec, full_spec],
        out_specs=[row_spec, full_spec, full_spec],
        out_shape=[
            jax.ShapeDtypeStruct((N, D), f32),
            jax.ShapeDtypeStruct((1, D), f32),
            jax.ShapeDtypeStruct((1, D), f32),
        ],
        compiler_params=pltpu.CompilerParams(
            dimension_semantics=("arbitrary",)),
    )(agg1, h1, dinv, b1r)

    h2 = pl.pallas_call(
        _bn_mm_body,
        grid=(GRID,),
        in_specs=[row_spec, full_spec, full_spec, full_spec, full_spec,
                  w_spec, vec_spec],
        out_specs=row_spec,
        out_shape=jax.ShapeDtypeStruct((N, D), f32),
    )(out1, s1, s2, gr, ber, W2, dinv)

    agg2 = _agg(packed_a, pads, h2, z2)

    out = pl.pallas_call(
        _final_body,
        grid=(GRID,),
        in_specs=[part_spec, row_spec, vec_spec, full_spec],
        out_specs=row_spec,
        out_shape=jax.ShapeDtypeStruct((N, D), f32),
    )(agg2, h2, dinv, b2r)

    return out


# speedup vs baseline: 1.0192x; 1.0192x over previous
import functools

import jax
import jax.numpy as jnp
from jax import lax
from jax.experimental import pallas as pl
from jax.experimental.pallas import tpu as pltpu
from jax.experimental.pallas import tpu_sc as plsc

N = 10000
E = 320000
D = 128
NC = 2
NS = 16
NW = NC * NS
NPAD = 10240
RPT = NPAD // NS
NBUF = 3
CHUNK = 96
K0 = 105
K1 = 105
EPAD = NS * (K0 + K1) * CHUNK
RB = 2000
GRID = N // RB

_mesh = plsc.VectorSubcoreMesh(core_axis_name="c", subcore_axis_name="s")


@functools.partial(
    pl.kernel,
    out_type=jax.ShapeDtypeStruct((NC, NPAD), jnp.float32),
    mesh=_mesh,
    scratch_types=[
        pltpu.VMEM((K0 * CHUNK,), jnp.int32),
        pltpu.VMEM((K0, CHUNK), jnp.int32),
        pltpu.VMEM((CHUNK,), jnp.float32),
        pltpu.VMEM_SHARED((NPAD,), jnp.float32),
        pltpu.SemaphoreType.DMA,
        pltpu.SemaphoreType.DMA,
    ],
)
def _deg(packed_hbm, pads_hbm, zeros_hbm, out_hbm,
         packed_v, dst2d, ones_v, acc, zsem, ssem):
    c = lax.axis_index("c")
    s = lax.axis_index("s")
    pltpu.async_copy(zeros_hbm, acc.at[pl.ds(s * RPT, RPT)], zsem)
    _stage_packed(packed_hbm, pads_hbm, packed_v, c, s, ssem)
    for i in range(CHUNK // 16):
        ones_v[pl.ds(i * 16, 16)] = jnp.ones((16,), jnp.float32)
    for j in range(K0):
        for i in range(CHUNK // 16):
            v = packed_v[pl.ds(j * CHUNK + i * 16, 16)]
            dst2d[j, pl.ds(i * 16, 16)] = lax.shift_right_logical(v, 16)
    pltpu.make_async_copy(zeros_hbm, acc.at[pl.ds(s * RPT, RPT)], zsem).wait()
    plsc.subcore_barrier()

    def fire(j, _):
        pltpu.async_copy(ones_v, acc.at[dst2d.at[j]], ssem, add=True)
        return ()

    lax.fori_loop(0, K0, fire, ())

    def drain(j, _):
        pltpu.make_async_copy(ones_v, acc.at[dst2d.at[j]], ssem).wait()
        return ()

    lax.fori_loop(0, K0, drain, ())
    plsc.subcore_barrier()
    pltpu.sync_copy(acc.at[pl.ds(s * RPT, RPT)],
                    out_hbm.at[c, pl.ds(s * RPT, RPT)])


def _stage_packed(packed_hbm, pads_hbm, packed_v, c, s, sem):
    n = K0 * CHUNK

    @pl.when(jnp.logical_or(c == 0, s < NS - 1))
    def _():
        base = jnp.where(c == 0, 0, NS * K0 * CHUNK)
        pltpu.async_copy(packed_hbm.at[pl.ds(base + s * n, n)],
                         packed_v.at[pl.ds(0, n)], sem)
        pltpu.make_async_copy(packed_hbm.at[pl.ds(base + s * n, n)],
                              packed_v.at[pl.ds(0, n)], sem).wait()

    @pl.when(jnp.logical_and(c == 1, s == NS - 1))
    def _():
        nreal = E - (2 * NS - 1) * n
        pltpu.async_copy(packed_hbm.at[pl.ds(E - nreal, nreal)],
                         packed_v.at[pl.ds(0, nreal)], sem)
        pltpu.make_async_copy(packed_hbm.at[pl.ds(E - nreal, nreal)],
                              packed_v.at[pl.ds(0, nreal)], sem).wait()
        pltpu.async_copy(pads_hbm, packed_v.at[pl.ds(nreal, EPAD - E)], sem)
        pltpu.make_async_copy(pads_hbm,
                              packed_v.at[pl.ds(nreal, EPAD - E)], sem).wait()


@functools.partial(
    pl.kernel,
    out_type=jax.ShapeDtypeStruct((NC, NPAD, D), jnp.float32),
    mesh=_mesh,
    scratch_types=(
        [pltpu.VMEM((K0 * CHUNK,), jnp.int32)]
        + [pltpu.VMEM((CHUNK,), jnp.int32) for _ in range(2 * NBUF)]
        + [pltpu.VMEM((CHUNK, D), jnp.float32) for _ in range(NBUF)]
        + [pltpu.VMEM_SHARED((NPAD, D), jnp.float32)]
        + [pltpu.SemaphoreType.DMA for _ in range(2 * NBUF)]
    ),
)
def _agg(packed_hbm, pads_hbm, h_hbm, zeros_hbm, out_hbm, packed_v, *rest):
    sidx = rest[0:NBUF]
    didx = rest[NBUF:2 * NBUF]
    bufs = rest[2 * NBUF:3 * NBUF]
    acc = rest[3 * NBUF]
    gsems = rest[3 * NBUF + 1:4 * NBUF + 1]
    ssems = rest[4 * NBUF + 1:5 * NBUF + 1]
    c = lax.axis_index("c")
    s = lax.axis_index("s")
    pltpu.async_copy(zeros_hbm, acc.at[pl.ds(s * RPT, RPT)], gsems[0])
    _stage_packed(packed_hbm, pads_hbm, packed_v, c, s, gsems[1])
    pltpu.make_async_copy(zeros_hbm, acc.at[pl.ds(s * RPT, RPT)],
                          gsems[0]).wait()
    plsc.subcore_barrier()

    def unpack(j, sb, db):
        for i in range(CHUNK // 16):
            v = packed_v[pl.ds(j * CHUNK + i * 16, 16)]
            sb[pl.ds(i * 16, 16)] = lax.bitwise_and(v, 0xFFFF)
            db[pl.ds(i * 16, 16)] = lax.shift_right_logical(v, 16)

    def run(nch):
        for b in range(NBUF):
            unpack(b, sidx[b], didx[b])
            pltpu.async_copy(h_hbm.at[sidx[b]], bufs[b], gsems[b])

        def outer(it, _):
            jj = it * NBUF
            for b in range(NBUF):
                pltpu.make_async_copy(h_hbm.at[sidx[b]], bufs[b],
                                      gsems[b]).wait()
                pltpu.async_copy(bufs[b], acc.at[didx[b]], ssems[b], add=True)
            for b in range(NBUF):
                nxt = jj + b + NBUF
                pltpu.make_async_copy(bufs[b], acc.at[didx[b]],
                                      ssems[b]).wait()

                @pl.when(nxt < nch)
                def _():
                    unpack(nxt, sidx[b], didx[b])
                    pltpu.async_copy(h_hbm.at[sidx[b]], bufs[b], gsems[b])

            return ()

        lax.fori_loop(0, nch // NBUF, outer, ())

    @pl.when(c == 0)
    def _():
        run(K0)

    @pl.when(c == 1)
    def _():
        run(K1)

    plsc.subcore_barrier()
    pltpu.sync_copy(acc.at[pl.ds(s * RPT, RPT)],
                    out_hbm.at[c, pl.ds(s * RPT, RPT)])


def _pack_body(e_ref, p_ref):
    e = e_ref[...]
    p_ref[...] = jnp.bitwise_or(e[0], e[1] << 16)


def _dinv_body(degp_ref, dinv_ref):
    dp = degp_ref[...]
    d = dp[:NPAD] + dp[NPAD:] + 1.0
    dinv_ref[...] = lax.rsqrt(d)


def _mm_scale_body(x_ref, w_ref, dinv_ref, o_ref):
    h = jnp.dot(x_ref[...], w_ref[...], preferred_element_type=jnp.float32)
    o_ref[...] = h * dinv_ref[...]


def _combine_stats_body(ap_ref, h_ref, dinv_ref, b_ref, o_ref, s1_ref, s2_ref):
    i = pl.program_id(0)
    ap = ap_ref[...]
    o = (ap[0] + ap[1] + h_ref[...]) * dinv_ref[...] + b_ref[...]
    o_ref[...] = o

    @pl.when(i == 0)
    def _():
        s1_ref[...] = jnp.zeros_like(s1_ref)
        s2_ref[...] = jnp.zeros_like(s2_ref)

    s1_ref[...] += jnp.sum(o, axis=0, keepdims=True)
    s2_ref[...] += jnp.sum(o * o, axis=0, keepdims=True)


def _bn_mm_body(o_ref, s1_ref, s2_ref, g_ref, be_ref, w_ref, dinv_ref, h_ref):
    mean = s1_ref[...] / N
    var = s2_ref[...] / N - mean * mean
    rstd = lax.rsqrt(var + 1e-5)
    y = (o_ref[...] - mean) * (rstd * g_ref[...]) + be_ref[...]
    y = jnp.maximum(y, 0.0)
    h = jnp.dot(y, w_ref[...], preferred_element_type=jnp.float32)
    h_ref[...] = h * dinv_ref[...]


def _final_body(ap_ref, h_ref, dinv_ref, b_ref, o_ref):
    ap = ap_ref[...]
    o_ref[...] = (ap[0] + ap[1] + h_ref[...]) * dinv_ref[...] + b_ref[...]


def kernel(x, edge_index, W1, b1, W2, b2, gamma, beta):
    f32 = jnp.float32
    npad_e = EPAD - E
    pad_src = jnp.arange(npad_e, dtype=jnp.int32) % N
    pad_dst = N + jnp.arange(npad_e, dtype=jnp.int32) % (NPAD - N)
    pads = jnp.bitwise_or(pad_src, pad_dst << 16)
    packed_a = pl.pallas_call(
        _pack_body,
        out_shape=jax.ShapeDtypeStruct((E // 256, 256), jnp.int32),
    )(edge_index.astype(jnp.int32).reshape(2, E // 256, 256)).reshape(E)
    z1 = jnp.zeros((RPT,), f32)
    z2 = jnp.zeros((RPT, D), f32)
    b1r = b1.reshape(1, D)
    b2r = b2.reshape(1, D)
    gr = gamma.reshape(1, D)
    ber = beta.reshape(1, D)

    deg_p = _deg(packed_a, pads, z1)

    dinv = pl.pallas_call(
        _dinv_body,
        out_shape=jax.ShapeDtypeStruct((NPAD, 1), f32),
    )(deg_p.reshape(NC * NPAD, 1))

    row_spec = pl.BlockSpec((RB, D), lambda i: (i, 0))
    vec_spec = pl.BlockSpec((RB, 1), lambda i: (i, 0))
    full_spec = pl.BlockSpec((1, D), lambda i: (0, 0))
    w_spec = pl.BlockSpec((D, D), lambda i: (0, 0))
    part_spec = pl.BlockSpec((NC, RB, D), lambda i: (0, i, 0))

    h1 = pl.pallas_call(
        _mm_scale_body,
        grid=(GRID,),
        in_specs=[row_spec, w_spec, vec_spec],
        out_specs=row_spec,
        out_shape=jax.ShapeDtypeStruct((N, D), f32),
    )(x, W1, dinv)

    agg1 = _agg(packed_a, pads, h1, z2)

    out1, s1, s2 = pl.pallas_call(
        _combine_stats_body,
        grid=(GRID,),
        in_specs=[part_spec, row_spec, vec_spec, full_spec],
        out_specs=[row_spec, full_spec, full_spec],
        out_shape=[
            jax.ShapeDtypeStruct((N, D), f32),
            jax.ShapeDtypeStruct((1, D), f32),
            jax.ShapeDtypeStruct((1, D), f32),
        ],
        compiler_params=pltpu.CompilerParams(
            dimension_semantics=("arbitrary",)),
    )(agg1, h1, dinv, b1r)

    h2 = pl.pallas_call(
        _bn_mm_body,
        grid=(GRID,),
        in_specs=[row_spec, full_spec, full_spec, full_spec, full_spec,
                  w_spec, vec_spec],
        out_specs=row_spec,
        out_shape=jax.ShapeDtypeStruct((N, D), f32),
    )(out1, s1, s2, gr, ber, W2, dinv)

    agg2 = _agg(packed_a, pads, h2, z2)

    out = pl.pallas_call(
        _final_body,
        grid=(GRID,),
        in_specs=[part_spec, row_spec, vec_spec, full_spec],
        out_specs=row_spec,
        out_shape=jax.ShapeDtypeStruct((N, D), f32),
    )(agg2, h2, dinv, b2r)

    return out


# speedup vs baseline: 1.0372x; 1.0176x over previous
import functools

import jax
import jax.numpy as jnp
from jax import lax
from jax.experimental import pallas as pl
from jax.experimental.pallas import tpu as pltpu
from jax.experimental.pallas import tpu_sc as plsc

N = 10000
E = 320000
D = 128
NC = 2
NS = 16
NW = NC * NS
NPAD = 10240
RPT = NPAD // NS
NBUF = 3
CHUNK = 96
K0 = 105
K1 = 105
EPAD = NS * (K0 + K1) * CHUNK
RB = 2000
GRID = N // RB

_mesh = plsc.VectorSubcoreMesh(core_axis_name="c", subcore_axis_name="s")


@functools.partial(
    pl.kernel,
    out_type=jax.ShapeDtypeStruct((NC, NPAD), jnp.float32),
    mesh=_mesh,
    scratch_types=[
        pltpu.VMEM((K0 * CHUNK,), jnp.int32),
        pltpu.VMEM((K0, CHUNK), jnp.int32),
        pltpu.VMEM((CHUNK,), jnp.float32),
        pltpu.VMEM_SHARED((NPAD,), jnp.float32),
        pltpu.SemaphoreType.DMA,
        pltpu.SemaphoreType.DMA,
    ],
)
def _deg(packed_hbm, pads_hbm, zeros_hbm, out_hbm,
         packed_v, dst2d, ones_v, acc, zsem, ssem):
    c = lax.axis_index("c")
    s = lax.axis_index("s")
    pltpu.async_copy(zeros_hbm, acc.at[pl.ds(s * RPT, RPT)], zsem)
    _stage_packed(packed_hbm, pads_hbm, packed_v, c, s, ssem)
    for i in range(CHUNK // 16):
        ones_v[pl.ds(i * 16, 16)] = jnp.ones((16,), jnp.float32)
    for j in range(K0):
        for i in range(CHUNK // 16):
            v = packed_v[pl.ds(j * CHUNK + i * 16, 16)]
            dst2d[j, pl.ds(i * 16, 16)] = lax.shift_right_logical(v, 16)
    pltpu.make_async_copy(zeros_hbm, acc.at[pl.ds(s * RPT, RPT)], zsem).wait()
    plsc.subcore_barrier()

    def fire(j, _):
        pltpu.async_copy(ones_v, acc.at[dst2d.at[j]], ssem, add=True)
        return ()

    lax.fori_loop(0, K0, fire, ())

    def drain(j, _):
        pltpu.make_async_copy(ones_v, acc.at[dst2d.at[j]], ssem).wait()
        return ()

    lax.fori_loop(0, K0, drain, ())
    plsc.subcore_barrier()
    pltpu.sync_copy(acc.at[pl.ds(s * RPT, RPT)],
                    out_hbm.at[c, pl.ds(s * RPT, RPT)])


def _stage_packed(packed_hbm, pads_hbm, packed_v, c, s, sem):
    n = K0 * CHUNK

    @pl.when(jnp.logical_or(c == 0, s < NS - 1))
    def _():
        base = jnp.where(c == 0, 0, NS * K0 * CHUNK)
        pltpu.async_copy(packed_hbm.at[pl.ds(base + s * n, n)],
                         packed_v.at[pl.ds(0, n)], sem)
        pltpu.make_async_copy(packed_hbm.at[pl.ds(base + s * n, n)],
                              packed_v.at[pl.ds(0, n)], sem).wait()

    @pl.when(jnp.logical_and(c == 1, s == NS - 1))
    def _():
        nreal = E - (2 * NS - 1) * n
        pltpu.async_copy(packed_hbm.at[pl.ds(E - nreal, nreal)],
                         packed_v.at[pl.ds(0, nreal)], sem)
        pltpu.make_async_copy(packed_hbm.at[pl.ds(E - nreal, nreal)],
                              packed_v.at[pl.ds(0, nreal)], sem).wait()
        pltpu.async_copy(pads_hbm, packed_v.at[pl.ds(nreal, EPAD - E)], sem)
        pltpu.make_async_copy(pads_hbm,
                              packed_v.at[pl.ds(nreal, EPAD - E)], sem).wait()


@functools.partial(
    pl.kernel,
    out_type=jax.ShapeDtypeStruct((NC, NPAD, D), jnp.float32),
    mesh=_mesh,
    scratch_types=(
        [pltpu.VMEM((K0 * CHUNK,), jnp.int32)]
        + [pltpu.VMEM((CHUNK,), jnp.int32) for _ in range(2 * NBUF)]
        + [pltpu.VMEM((CHUNK, D), jnp.float32) for _ in range(NBUF)]
        + [pltpu.VMEM_SHARED((NPAD, D), jnp.float32)]
        + [pltpu.SemaphoreType.DMA for _ in range(2 * NBUF)]
    ),
)
def _agg(packed_hbm, pads_hbm, h_hbm, zeros_hbm, out_hbm, packed_v, *rest):
    sidx = rest[0:NBUF]
    didx = rest[NBUF:2 * NBUF]
    bufs = rest[2 * NBUF:3 * NBUF]
    acc = rest[3 * NBUF]
    gsems = rest[3 * NBUF + 1:4 * NBUF + 1]
    ssems = rest[4 * NBUF + 1:5 * NBUF + 1]
    c = lax.axis_index("c")
    s = lax.axis_index("s")
    pltpu.async_copy(zeros_hbm, acc.at[pl.ds(s * RPT, RPT)], gsems[0])
    _stage_packed(packed_hbm, pads_hbm, packed_v, c, s, gsems[1])
    pltpu.make_async_copy(zeros_hbm, acc.at[pl.ds(s * RPT, RPT)],
                          gsems[0]).wait()
    plsc.subcore_barrier()

    def unpack(j, sb, db):
        for i in range(CHUNK // 16):
            v = packed_v[pl.ds(j * CHUNK + i * 16, 16)]
            sb[pl.ds(i * 16, 16)] = lax.bitwise_and(v, 0xFFFF)
            db[pl.ds(i * 16, 16)] = lax.shift_right_logical(v, 16)

    def run(nch):
        for b in range(NBUF):
            unpack(b, sidx[b], didx[b])
            pltpu.async_copy(h_hbm.at[sidx[b]], bufs[b], gsems[b])

        def outer(it, _):
            jj = it * NBUF
            for b in range(NBUF):
                pltpu.make_async_copy(h_hbm.at[sidx[b]], bufs[b],
                                      gsems[b]).wait()
                pltpu.async_copy(bufs[b], acc.at[didx[b]], ssems[b], add=True)
            for b in range(NBUF):
                nxt = jj + b + NBUF
                pltpu.make_async_copy(bufs[b], acc.at[didx[b]],
                                      ssems[b]).wait()

                @pl.when(nxt < nch)
                def _():
                    unpack(nxt, sidx[b], didx[b])
                    pltpu.async_copy(h_hbm.at[sidx[b]], bufs[b], gsems[b])

            return ()

        lax.fori_loop(0, nch // NBUF, outer, ())

    @pl.when(c == 0)
    def _():
        run(K0)

    @pl.when(c == 1)
    def _():
        run(K1)

    plsc.subcore_barrier()
    pltpu.sync_copy(acc.at[pl.ds(s * RPT, RPT)],
                    out_hbm.at[c, pl.ds(s * RPT, RPT)])


def _pack_body(e_ref, p_ref):
    e = e_ref[...]
    p_ref[...] = jnp.bitwise_or(e[0], e[1] << 16)


def _dinv_body(degp_ref, dinv_ref):
    dp = degp_ref[...]
    d = dp[:NPAD] + dp[NPAD:] + 1.0
    dinv_ref[...] = lax.rsqrt(d)


def _mm_scale_body(x_ref, w_ref, dinv_ref, o_ref):
    h = jnp.dot(x_ref[...], w_ref[...], preferred_element_type=jnp.float32)
    o_ref[...] = h * dinv_ref[...]


def _mid_body(ap_ref, h_ref, dinv_ref, b_ref, g_ref, be_ref, w_ref,
              h2_ref, o_s, s1_s, s2_s):
    p = pl.program_id(0)
    i = pl.program_id(1)

    @pl.when(p == 0)
    def _():
        ap = ap_ref[...]
        o = (ap[0] + ap[1] + h_ref[...]) * dinv_ref[...] + b_ref[...]
        o_s[pl.ds(i * RB, RB), :] = o

        @pl.when(i == 0)
        def _():
            s1_s[...] = jnp.zeros_like(s1_s)
            s2_s[...] = jnp.zeros_like(s2_s)

        s1_s[...] += jnp.sum(o, axis=0, keepdims=True)
        s2_s[...] += jnp.sum(o * o, axis=0, keepdims=True)

    @pl.when(p == 1)
    def _():
        mean = s1_s[...] / N
        var = s2_s[...] / N - mean * mean
        rstd = lax.rsqrt(var + 1e-5)
        y = (o_s[pl.ds(i * RB, RB), :] - mean) * (rstd * g_ref[...]) + be_ref[...]
        y = jnp.maximum(y, 0.0)
        h = jnp.dot(y, w_ref[...], preferred_element_type=jnp.float32)
        h2_ref[...] = h * dinv_ref[...]


def _final_body(ap_ref, h_ref, dinv_ref, b_ref, o_ref):
    ap = ap_ref[...]
    o_ref[...] = (ap[0] + ap[1] + h_ref[...]) * dinv_ref[...] + b_ref[...]


def kernel(x, edge_index, W1, b1, W2, b2, gamma, beta):
    f32 = jnp.float32
    npad_e = EPAD - E
    pad_src = jnp.arange(npad_e, dtype=jnp.int32) % N
    pad_dst = N + jnp.arange(npad_e, dtype=jnp.int32) % (NPAD - N)
    pads = jnp.bitwise_or(pad_src, pad_dst << 16)
    packed_a = pl.pallas_call(
        _pack_body,
        out_shape=jax.ShapeDtypeStruct((E // 256, 256), jnp.int32),
    )(edge_index.astype(jnp.int32).reshape(2, E // 256, 256)).reshape(E)
    z1 = jnp.zeros((RPT,), f32)
    z2 = jnp.zeros((RPT, D), f32)
    b1r = b1.reshape(1, D)
    b2r = b2.reshape(1, D)
    gr = gamma.reshape(1, D)
    ber = beta.reshape(1, D)

    deg_p = _deg(packed_a, pads, z1)

    dinv = pl.pallas_call(
        _dinv_body,
        out_shape=jax.ShapeDtypeStruct((NPAD, 1), f32),
    )(deg_p.reshape(NC * NPAD, 1))

    row_spec = pl.BlockSpec((RB, D), lambda i: (i, 0))
    vec_spec = pl.BlockSpec((RB, 1), lambda i: (i, 0))
    full_spec = pl.BlockSpec((1, D), lambda i: (0, 0))
    w_spec = pl.BlockSpec((D, D), lambda i: (0, 0))
    part_spec = pl.BlockSpec((NC, RB, D), lambda i: (0, i, 0))

    h1 = pl.pallas_call(
        _mm_scale_body,
        grid=(GRID,),
        in_specs=[row_spec, w_spec, vec_spec],
        out_specs=row_spec,
        out_shape=jax.ShapeDtypeStruct((N, D), f32),
    )(x, W1, dinv)

    agg1 = _agg(packed_a, pads, h1, z2)

    h2 = pl.pallas_call(
        _mid_body,
        grid=(2, GRID),
        in_specs=[
            pl.BlockSpec((NC, RB, D),
                         lambda p, i: (0, jnp.where(p == 0, i, 0), 0)),
            pl.BlockSpec((RB, D), lambda p, i: (jnp.where(p == 0, i, 0), 0)),
            pl.BlockSpec((RB, 1), lambda p, i: (i, 0)),
            pl.BlockSpec((1, D), lambda p, i: (0, 0)),
            pl.BlockSpec((1, D), lambda p, i: (0, 0)),
            pl.BlockSpec((1, D), lambda p, i: (0, 0)),
            pl.BlockSpec((D, D), lambda p, i: (0, 0)),
        ],
        out_specs=pl.BlockSpec((RB, D),
                               lambda p, i: (jnp.where(p == 0, 0, i), 0)),
        out_shape=jax.ShapeDtypeStruct((N, D), f32),
        scratch_shapes=[
            pltpu.VMEM((N, D), f32),
            pltpu.VMEM((1, D), f32),
            pltpu.VMEM((1, D), f32),
        ],
        compiler_params=pltpu.CompilerParams(
            dimension_semantics=("arbitrary", "arbitrary")),
    )(agg1, h1, dinv, b1r, gr, ber, W2)

    agg2 = _agg(packed_a, pads, h2, z2)

    out = pl.pallas_call(
        _final_body,
        grid=(GRID,),
        in_specs=[part_spec, row_spec, vec_spec, full_spec],
        out_specs=row_spec,
        out_shape=jax.ShapeDtypeStruct((N, D), f32),
    )(agg2, h2, dinv, b2r)

    return out
